# Initial kernel scaffold; baseline (speedup 1.0000x reference)
import jax
import jax.numpy as jnp
from jax.experimental import pallas as pl


def kernel(x, edge_index, batch, num_graphs, params):
    raise NotImplementedError("write your pallas kernel here")


# speedup vs baseline: 25.1176x; 25.1176x over previous
import functools

import numpy as np
import jax
import jax.numpy as jnp
from jax import lax
from jax.experimental import pallas as pl
from jax.experimental.pallas import tpu as pltpu
from jax.experimental.pallas import tpu_sc as plsc

NF = 128
NG = 256
N = NG * NF
E = N * 16
HC = 32
NL = 3
HID = 64
NCLS = 10
IDIM = NF * (NF - 1) // 2
KDIM = NF * NF
SB = float(1.0 / np.sqrt(1.0 + 1e-5))

NCORES = 2
NSUB = 16
NW = NCORES * NSUB

F32 = jnp.float32

_iu0, _iu1 = np.triu_indices(NF, k=1)
_TRIU_MAP = np.full((KDIM,), IDIM, dtype=np.int32)
_TRIU_MAP[_iu0 * NF + _iu1] = np.arange(IDIM, dtype=np.int32)

_MESH = dict(core_axis_name="c", subcore_axis_name="s",
             num_cores=NCORES, num_subcores=NSUB)
_SC_PARAMS = pltpu.CompilerParams(use_tc_tiling_on_sc=False)


def _wid():
    return lax.axis_index("c") * NSUB + lax.axis_index("s")


def _fill(buf, rows, width, value):
    def body(i, _):
        for j in range(width // 16):
            buf[i, pl.ds(j * 16, 16)] = jnp.full((16,), value, F32)
        return 0
    lax.fori_loop(0, rows, body, 0)


def _zero_table(zbuf, table, row0, nrows):
    def body(i, _):
        pltpu.sync_copy(zbuf, table.at[pl.ds(row0 + i * 128, 128)])
        return 0
    lax.fori_loop(0, nrows // 128, body, 0)


def _sc_pass_a(dst2d, batch2d, map2d, w1pad):
    out_type = (
        jax.ShapeDtypeStruct((NCORES, N, 16), F32),
        jax.ShapeDtypeStruct((NCORES, NG, 16), F32),
        jax.ShapeDtypeStruct((KDIM, HID), F32),
    )
    scratch = [
        pltpu.VMEM((128, 16), F32),
        pltpu.VMEM((128, 16), F32),
        pltpu.VMEM((128,), jnp.int32),
        pltpu.VMEM((128,), jnp.int32),
        pltpu.VMEM((128, HID), F32),
        pltpu.VMEM_SHARED((N, 16), F32),
        pltpu.VMEM_SHARED((NG, 16), F32),
        pltpu.SemaphoreType.DMA,
    ]

    @functools.partial(
        pl.kernel, out_type=out_type,
        mesh=plsc.VectorSubcoreMesh(**_MESH), scratch_types=scratch,
        compiler_params=_SC_PARAMS)
    def k(dst_h, batch_h, map_h, w1_h, deg_o, cnt_o, t_o,
          ones_v, zbuf, idx_v, midx_v, trow_v, deg_t, cnt_t, sem):
        c = lax.axis_index("c")
        s = lax.axis_index("s")
        w = _wid()
        _fill(ones_v, 128, 16, 1.0)
        _fill(zbuf, 128, 16, 0.0)
        _zero_table(zbuf, deg_t, s * (N // NSUB), N // NSUB)

        @pl.when(s == 0)
        def _():
            _zero_table(zbuf, cnt_t, 0, NG)

        plsc.subcore_barrier()

        def ehist(i, _):
            pltpu.sync_copy(dst_h.at[w * 128 + i], idx_v)
            pltpu.sync_copy(ones_v, deg_t.at[idx_v], add=True)
            return 0
        lax.fori_loop(0, 128, ehist, 0)

        def bhist(i, _):
            pltpu.sync_copy(batch_h.at[w * 8 + i], idx_v)
            pltpu.sync_copy(ones_v, cnt_t.at[idx_v], add=True)
            return 0
        lax.fori_loop(0, 8, bhist, 0)

        for i in range(4):
            row = w * 4 + i
            pltpu.sync_copy(map_h.at[row], midx_v)
            pltpu.async_copy(w1_h.at[midx_v], trow_v, sem).wait()
            pltpu.sync_copy(trow_v, t_o.at[pl.ds(row * 128, 128)])

        plsc.subcore_barrier()

        def dout(i, _):
            r = s * (N // NSUB) + i * 128
            pltpu.sync_copy(deg_t.at[pl.ds(r, 128)], deg_o.at[c, pl.ds(r, 128)])
            return 0
        lax.fori_loop(0, (N // NSUB) // 128, dout, 0)

        @pl.when(s == 0)
        def _():
            for i in range(NG // 128):
                pltpu.sync_copy(cnt_t.at[pl.ds(i * 128, 128)],
                                cnt_o.at[c, pl.ds(i * 128, 128)])

    return k(dst2d, batch2d, map2d, w1pad)


_SS = 8


def _sc_scatter(src2d, dst2d, y):
    out_type = jax.ShapeDtypeStruct((NCORES, N, HC), F32)
    scratch = [
        pltpu.VMEM((128, HC), F32),
        pltpu.VMEM((_SS, 128), jnp.int32),
        pltpu.VMEM((_SS, 128), jnp.int32),
        pltpu.VMEM((_SS, 128, HC), F32),
        pltpu.VMEM_SHARED((N, HC), F32),
        pltpu.SemaphoreType.DMA,
    ]

    @functools.partial(
        pl.kernel, out_type=out_type,
        mesh=plsc.VectorSubcoreMesh(**_MESH), scratch_types=scratch,
        compiler_params=_SC_PARAMS)
    def k(src_h, dst_h, y_h, acc_o, zbuf, sidx, didx, rows, acc_t, sem):
        c = lax.axis_index("c")
        s = lax.axis_index("s")
        w = _wid()
        _fill(zbuf, 128, HC, 0.0)
        _zero_table(zbuf, acc_t, s * (N // NSUB), N // NSUB)
        plsc.subcore_barrier()

        nsteps = (E // 128) // NW // _SS

        def step(t, _):
            row0 = w * (nsteps * _SS) + t * _SS
            pltpu.sync_copy(src_h.at[pl.ds(row0, _SS)], sidx)
            pltpu.sync_copy(dst_h.at[pl.ds(row0, _SS)], didx)
            descs = [pltpu.async_copy(y_h.at[sidx.at[j]], rows.at[j], sem)
                     for j in range(_SS)]
            for j in range(_SS):
                descs[j].wait()
            for j in range(_SS):
                pltpu.sync_copy(rows.at[j], acc_t.at[didx.at[j]], add=True)
            return 0
        lax.fori_loop(0, nsteps, step, 0)

        plsc.subcore_barrier()

        def aout(i, _):
            r = s * (N // NSUB) + i * 128
            pltpu.sync_copy(acc_t.at[pl.ds(r, 128)], acc_o.at[c, pl.ds(r, 128)])
            return 0
        lax.fori_loop(0, (N // NSUB) // 128, aout, 0)

    return k(src2d, dst2d, y)


def _sc_segsum(batch2d, x1, x2, x3):
    out_type = jax.ShapeDtypeStruct((NCORES, NL, NG, HC), F32)
    scratch = [
        pltpu.VMEM((128, HC), F32),
        pltpu.VMEM((128,), jnp.int32),
        pltpu.VMEM((128, HC), F32),
        pltpu.VMEM((128, HC), F32),
        pltpu.VMEM((128, HC), F32),
        pltpu.VMEM_SHARED((NG, HC), F32),
        pltpu.VMEM_SHARED((NG, HC), F32),
        pltpu.VMEM_SHARED((NG, HC), F32),
    ]

    @functools.partial(
        pl.kernel, out_type=out_type,
        mesh=plsc.VectorSubcoreMesh(**_MESH), scratch_types=scratch,
        compiler_params=_SC_PARAMS)
    def k(batch_h, x1_h, x2_h, x3_h, s_o,
          zbuf, bidx, r1, r2, r3, st1, st2, st3):
        c = lax.axis_index("c")
        s = lax.axis_index("s")
        w = _wid()
        _fill(zbuf, 128, HC, 0.0)

        @pl.when(s == 0)
        def _():
            for tab in (st1, st2, st3):
                _zero_table(zbuf, tab, 0, NG)

        plsc.subcore_barrier()

        def step(i, _):
            base = w * (N // NW) + i * 128
            pltpu.sync_copy(batch_h.at[base // 128], bidx)
            pltpu.sync_copy(x1_h.at[pl.ds(base, 128)], r1)
            pltpu.sync_copy(x2_h.at[pl.ds(base, 128)], r2)
            pltpu.sync_copy(x3_h.at[pl.ds(base, 128)], r3)
            pltpu.sync_copy(r1, st1.at[bidx], add=True)
            pltpu.sync_copy(r2, st2.at[bidx], add=True)
            pltpu.sync_copy(r3, st3.at[bidx], add=True)
            return 0
        lax.fori_loop(0, (N // NW) // 128, step, 0)

        plsc.subcore_barrier()

        @pl.when(s == 0)
        def _():
            for j, tab in enumerate((st1, st2, st3)):
                for i in range(NG // 128):
                    pltpu.sync_copy(tab.at[pl.ds(i * 128, 128)],
                                    s_o.at[c, j, pl.ds(i * 128, 128)])

    return k(batch2d, x1, x2, x3)


_R = 2048


def _tc_k1(deg, x, w1):
    def body(deg_ref, x_ref, w_ref, dinv_ref, y_ref):
        d = deg_ref[0, :, 0:1] + deg_ref[1, :, 0:1] + 1.0
        dv = lax.rsqrt(d)
        dinv_ref[...] = dv
        y_ref[...] = jnp.dot(x_ref[...], w_ref[...],
                             preferred_element_type=F32) * dv

    return pl.pallas_call(
        body,
        grid=(N // _R,),
        in_specs=[
            pl.BlockSpec((NCORES, _R, 16), lambda r: (0, r, 0)),
            pl.BlockSpec((_R, NF), lambda r: (r, 0)),
            pl.BlockSpec((NF, HC), lambda r: (0, 0)),
        ],
        out_specs=[
            pl.BlockSpec((_R, 1), lambda r: (r, 0)),
            pl.BlockSpec((_R, HC), lambda r: (r, 0)),
        ],
        out_shape=[
            jax.ShapeDtypeStruct((N, 1), F32),
            jax.ShapeDtypeStruct((N, HC), F32),
        ],
    )(deg, x, w1)


def _tc_combine(acc, y, dinv, b, w_next):
    def body(acc_ref, y_ref, dinv_ref, b_ref, w_ref, xn_ref, yn_ref):
        a = acc_ref[0] + acc_ref[1] + y_ref[...]
        dv = dinv_ref[...]
        o = jnp.tanh(a * dv + b_ref[...])
        xn_ref[...] = o
        yn_ref[...] = jnp.dot(o, w_ref[...], preferred_element_type=F32) * dv

    return pl.pallas_call(
        body,
        grid=(N // _R,),
        in_specs=[
            pl.BlockSpec((NCORES, _R, HC), lambda r: (0, r, 0)),
            pl.BlockSpec((_R, HC), lambda r: (r, 0)),
            pl.BlockSpec((_R, 1), lambda r: (r, 0)),
            pl.BlockSpec((1, HC), lambda r: (0, 0)),
            pl.BlockSpec((HC, HC), lambda r: (0, 0)),
        ],
        out_specs=[
            pl.BlockSpec((_R, HC), lambda r: (r, 0)),
            pl.BlockSpec((_R, HC), lambda r: (r, 0)),
        ],
        out_shape=[
            jax.ShapeDtypeStruct((N, HC), F32),
            jax.ShapeDtypeStruct((N, HC), F32),
        ],
    )(acc, y, dinv, b, w_next)


def _tc_combine_last(acc, y, dinv, b):
    def body(acc_ref, y_ref, dinv_ref, b_ref, xn_ref):
        a = acc_ref[0] + acc_ref[1] + y_ref[...]
        xn_ref[...] = jnp.tanh(a * dinv_ref[...] + b_ref[...])

    return pl.pallas_call(
        body,
        grid=(N // _R,),
        in_specs=[
            pl.BlockSpec((NCORES, _R, HC), lambda r: (0, r, 0)),
            pl.BlockSpec((_R, HC), lambda r: (r, 0)),
            pl.BlockSpec((_R, 1), lambda r: (r, 0)),
            pl.BlockSpec((1, HC), lambda r: (0, 0)),
        ],
        out_specs=pl.BlockSpec((_R, HC), lambda r: (r, 0)),
        out_shape=jax.ShapeDtypeStruct((N, HC), F32),
    )(acc, y, dinv, b)


_KB = 2048


def _tc_mlp(xr, t, seg, cnt, w1bp, g1s, t1, w2p, t2, w3p, t3, w4, b4):
    nk = KDIM // _KB

    def body(xr_ref, t_ref, s_ref, cnt_ref, w1bp_ref, g1s_ref, t1_ref,
             w2_ref, t2_ref, w3_ref, t3_ref, w4_ref, b4_ref, z_ref, accv):
        kk = pl.program_id(0)

        @pl.when(kk == 0)
        def _():
            accv[...] = jnp.zeros((NG, HID), F32)

        accv[...] += jnp.dot(xr_ref[...], t_ref[...],
                             preferred_element_type=F32)

        @pl.when(kk == nk - 1)
        def _():
            cm = jnp.maximum(cnt_ref[0, :, 0:1] + cnt_ref[1, :, 0:1], 1.0)
            a = accv[...]
            for j in range(NL):
                hj = (s_ref[0, j] + s_ref[1, j]) / cm
                a = a + jnp.dot(hj, w1bp_ref[j], preferred_element_type=F32)
            z1 = jnp.maximum(a * g1s_ref[...] + t1_ref[...], 0.0)
            z2 = jnp.maximum(
                jnp.dot(z1, w2_ref[...], preferred_element_type=F32)
                + t2_ref[...], 0.0)
            z3 = jnp.maximum(
                jnp.dot(z2, w3_ref[...], preferred_element_type=F32)
                + t3_ref[...], 0.0)
            z_ref[...] = (jnp.dot(z3, w4_ref[...], preferred_element_type=F32)
                          + b4_ref[...])

    return pl.pallas_call(
        body,
        grid=(nk,),
        in_specs=[
            pl.BlockSpec((NG, _KB), lambda k: (0, k)),
            pl.BlockSpec((_KB, HID), lambda k: (k, 0)),
            pl.BlockSpec((NCORES, NL, NG, HC), lambda k: (0, 0, 0, 0)),
            pl.BlockSpec((NCORES, NG, 16), lambda k: (0, 0, 0)),
            pl.BlockSpec((NL, HC, HID), lambda k: (0, 0, 0)),
            pl.BlockSpec((1, HID), lambda k: (0, 0)),
            pl.BlockSpec((1, HID), lambda k: (0, 0)),
            pl.BlockSpec((HID, HC), lambda k: (0, 0)),
            pl.BlockSpec((1, HC), lambda k: (0, 0)),
            pl.BlockSpec((HC, HC), lambda k: (0, 0)),
            pl.BlockSpec((1, HC), lambda k: (0, 0)),
            pl.BlockSpec((HC, NCLS), lambda k: (0, 0)),
            pl.BlockSpec((1, NCLS), lambda k: (0, 0)),
        ],
        out_specs=pl.BlockSpec((NG, NCLS), lambda k: (0, 0)),
        out_shape=jax.ShapeDtypeStruct((NG, NCLS), F32),
        scratch_shapes=[pltpu.VMEM((NG, HID), F32)],
    )(xr, t, seg, cnt, w1bp, g1s, t1, w2p, t2, w3p, t3, w4, b4)


def kernel(x, edge_index, batch, num_graphs, params):
    x = x.astype(F32)
    src2d = edge_index[0].reshape(E // 128, 128)
    dst2d = edge_index[1].reshape(E // 128, 128)
    batch2d = batch.reshape(N // 128, 128)
    map2d = jnp.asarray(_TRIU_MAP).reshape(KDIM // 128, 128)

    (w1c, b1c), (w2c, b2c), (w3c, b3c) = params['convs']
    g_bn, b_bn = params['bn']
    gh, bh = params['bnh']
    m = params['mlp']

    w1a = m['W1'][:IDIM]
    w1b = m['W1'][IDIM:]
    w1pad = jnp.concatenate(
        [w1a * (SB * g_bn)[:, None], jnp.zeros((1, HID), F32)], axis=0)
    w1bp = (w1b * (SB * gh)[:, None]).reshape(NL, HC, HID)
    c01 = b_bn @ w1a + bh @ w1b + m['b1']
    g1s = (SB * m['g1'])[None]
    t1 = (c01 * SB * m['g1'] + m['be1'])[None]
    w2p = m['W2'] * (SB * m['g2'])[None, :]
    t2 = (m['b2'] * SB * m['g2'] + m['be2'])[None]
    w3p = m['W3'] * (SB * m['g3'])[None, :]
    t3 = (m['b3'] * SB * m['g3'] + m['be3'])[None]
    b4 = m['b4'][None]

    deg, cnt, t_mat = _sc_pass_a(dst2d, batch2d, map2d, w1pad)

    dinv, y1 = _tc_k1(deg, x, w1c)
    acc1 = _sc_scatter(src2d, dst2d, y1)
    x1, y2 = _tc_combine(acc1, y1, dinv, b1c[None], w2c)
    acc2 = _sc_scatter(src2d, dst2d, y2)
    x2, y3 = _tc_combine(acc2, y2, dinv, b2c[None], w3c)
    acc3 = _sc_scatter(src2d, dst2d, y3)
    x3 = _tc_combine_last(acc3, y3, dinv, b3c[None])

    seg = _sc_segsum(batch2d, x1, x2, x3)

    xr = x.reshape(NG, KDIM)
    return _tc_mlp(xr, t_mat, seg, cnt, w1bp, g1s, t1,
                   w2p, t2, w3p, t3, m['W4'], b4)
